# Initial kernel scaffold; baseline (speedup 1.0000x reference)
import jax
import jax.numpy as jnp
from jax.experimental import pallas as pl


def kernel(h, e, edge_index, params):
    raise NotImplementedError("write your pallas kernel here")


# speedup vs baseline: 1.3430x; 1.3430x over previous
import functools

import jax
import jax.numpy as jnp
from jax import lax
from jax.experimental import pallas as pl
from jax.experimental.pallas import tpu as pltpu
from jax.experimental.pallas import tpu_sc as plsc

N = 10000
E = 320000
D = 128
ED = 16
NC = 2
NS = 16
NW = NC * NS
EPW = E // NW
C = 40
NCHUNK = EPW // C
CD = 80
NCHUNK_D = EPW // CD
NROW = 10112
NPS = NROW // NS
NPAD = 10240

F32 = jnp.float32


def _sds(shape, dtype=F32):
    return jax.ShapeDtypeStruct(shape, dtype)


def _tc_node_pre(h, pw1, wh1, wh2, qw1):
    def body(h_ref, pw1_ref, wh1_ref, wh2_ref, qw1_ref, hp, h1, h2, hq):
        hb = h_ref[...]
        hp[...] = jnp.dot(hb, pw1_ref[...], preferred_element_type=F32)
        h1[...] = jnp.dot(hb, wh1_ref[...], preferred_element_type=F32)
        h2[...] = jnp.dot(hb, wh2_ref[...], preferred_element_type=F32)
        hq[...] = jnp.dot(hb, qw1_ref[...], preferred_element_type=F32)

    g = 10
    blk = N // g
    full = lambda s: pl.BlockSpec(s, lambda i: (0, 0))
    return pl.pallas_call(
        body,
        grid=(g,),
        in_specs=[
            pl.BlockSpec((blk, D), lambda i: (i, 0)),
            full((D, D)), full((D, ED)), full((D, ED)), full((D, D)),
        ],
        out_specs=[
            pl.BlockSpec((blk, D), lambda i: (i, 0)),
            pl.BlockSpec((blk, ED), lambda i: (i, 0)),
            pl.BlockSpec((blk, ED), lambda i: (i, 0)),
            pl.BlockSpec((blk, D), lambda i: (i, 0)),
        ],
        out_shape=[_sds((N, D)), _sds((N, ED)), _sds((N, ED)), _sds((N, D))],
    )(h, pw1, wh1, wh2, qw1)


def _tc_edge_pre(e, pw2, pb, we, wb):
    def body(e_ref, pw2_ref, pb_ref, we_ref, wb_ref, ep, ee):
        eb = e_ref[...]
        ep[...] = jnp.dot(eb, pw2_ref[...], preferred_element_type=F32) + pb_ref[...]
        ee[...] = jnp.dot(eb, we_ref[...], preferred_element_type=F32) + wb_ref[...]

    g = 80
    blk = E // g
    full = lambda s: pl.BlockSpec(s, lambda i: (0, 0))
    return pl.pallas_call(
        body,
        grid=(g,),
        in_specs=[
            pl.BlockSpec((blk, ED), lambda i: (i, 0)),
            full((ED, D)), full((1, D)), full((ED, ED)), full((1, ED)),
        ],
        out_specs=[
            pl.BlockSpec((blk, D), lambda i: (i, 0)),
            pl.BlockSpec((blk, ED), lambda i: (i, 0)),
        ],
        out_shape=[_sds((E, D)), _sds((E, ED))],
    )(e, pw2, pb, we, wb)


def _tc_node_update(hq, agga, aggb, deg_all, qw2, qb):
    def body(hq_ref, aa_ref, ab_ref, deg_ref, qw2_ref, qb_ref, out):
        deg = jnp.sum(deg_ref[...], axis=0)
        agg = (aa_ref[...] + ab_ref[...]) / deg
        out[...] = jnp.maximum(
            hq_ref[...]
            + jnp.dot(agg, qw2_ref[...], preferred_element_type=F32)
            + qb_ref[...], 0.0)

    g = 10
    blk = N // g
    full = lambda s: pl.BlockSpec(s, lambda i: (0, 0))
    return pl.pallas_call(
        body,
        grid=(g,),
        in_specs=[
            pl.BlockSpec((blk, D), lambda i: (i, 0)),
            pl.BlockSpec((blk, D), lambda i: (i, 0)),
            pl.BlockSpec((blk, D), lambda i: (i, 0)),
            pl.BlockSpec((NW, blk, 1), lambda i: (0, i, 0)),
            full((D, D)), full((1, D)),
        ],
        out_specs=pl.BlockSpec((blk, D), lambda i: (i, 0)),
        out_shape=_sds((N, D)),
    )(hq, agga, aggb, deg_all, qw2, qb)


def _tc_heads_node(h, a0, b0c, bb0, n0w, n0b, n1w, n1b):
    def body(h_ref, a0_ref, b0_ref, bb0_ref, n0w_ref, n0b_ref, n1w_ref,
             n1b_ref, a_out, b_out, np_out):
        hb = h_ref[...]
        a_out[...] = jnp.dot(hb, a0_ref[...], preferred_element_type=F32) + bb0_ref[...]
        b_out[...] = jnp.dot(hb, b0_ref[...], preferred_element_type=F32)
        hid = jnp.maximum(
            jnp.dot(hb, n0w_ref[...], preferred_element_type=F32) + n0b_ref[...], 0.0)
        np_out[...] = jnp.dot(hid, n1w_ref[...], preferred_element_type=F32) + n1b_ref[...]

    g = 10
    blk = N // g
    full = lambda s: pl.BlockSpec(s, lambda i: (0, 0))
    return pl.pallas_call(
        body,
        grid=(g,),
        in_specs=[
            pl.BlockSpec((blk, D), lambda i: (i, 0)),
            full((D, 64)), full((D, 64)), full((1, 64)),
            full((D, 64)), full((1, 64)), full((64, 1)), full((1, 1)),
        ],
        out_specs=[
            pl.BlockSpec((blk, 64), lambda i: (i, 0)),
            pl.BlockSpec((blk, 64), lambda i: (i, 0)),
            pl.BlockSpec((blk, 1), lambda i: (i, 0)),
        ],
        out_shape=[_sds((N, 64)), _sds((N, 64)), _sds((N, 1))],
    )(h, a0, b0c, bb0, n0w, n0b, n1w, n1b)


def _tc_edge_head(r, w1, b1):
    def body(r_ref, w1_ref, b1_ref, out):
        out[...] = jnp.dot(r_ref[...], w1_ref[...], preferred_element_type=F32) + b1_ref[...]

    g = 80
    blk = E // g
    full = lambda s: pl.BlockSpec(s, lambda i: (0, 0))
    return pl.pallas_call(
        body,
        grid=(g,),
        in_specs=[
            pl.BlockSpec((blk, 64), lambda i: (i, 0)),
            full((64, 1)), full((1, 1)),
        ],
        out_specs=pl.BlockSpec((blk, 1), lambda i: (i, 0)),
        out_shape=_sds((E, 1)),
    )(r, w1, b1)


_MESH = plsc.VectorSubcoreMesh(core_axis_name="c", subcore_axis_name="s")


def _sc_degree(dst):
    @functools.partial(
        pl.kernel,
        out_type=_sds((NW * (NPAD // 16), 16)),
        mesh=_MESH,
        compiler_params=pltpu.CompilerParams(use_tc_tiling_on_sc=False, needs_layout_passes=False),
        scratch_types=[
            pltpu.VMEM((NPAD // 16, 16), F32),
            pltpu.VMEM((CD,), jnp.int32),
        ],
    )
    def k(dst_hbm, out_hbm, hist_v, idx_v):
        cid = lax.axis_index("c")
        sid = lax.axis_index("s")
        wid = cid * NS + sid
        zero16 = jnp.zeros((16,), F32)

        def zbody(r, _):
            hist_v[r, :] = zero16
            return 0
        lax.fori_loop(0, NPAD // 16, zbody, 0)

        ones16 = jnp.ones((16,), F32)

        def chunk(c, _):
            base = wid * EPW + c * CD
            pltpu.sync_copy(dst_hbm.at[pl.ds(base, CD)], idx_v)
            for kk in range(CD // 16):
                iv = idx_v[pl.ds(kk * 16, 16)]
                rows = lax.shift_right_logical(iv, 4)
                cols = lax.bitwise_and(iv, 15)
                plsc.addupdate_scatter(hist_v, [rows, cols], ones16)
            return 0
        lax.fori_loop(0, NCHUNK_D, chunk, 0)

        pltpu.sync_copy(hist_v, out_hbm.at[pl.ds(wid * (NPAD // 16), NPAD // 16)])

    return k(dst)


def _sc_layer(hp, ep, h1, h2, ee, srcv, dstv, zeros_nd):
    @functools.partial(
        pl.kernel,
        out_type=(_sds((2 * NROW, D)), _sds((E, ED))),
        mesh=_MESH,
        compiler_params=pltpu.CompilerParams(use_tc_tiling_on_sc=False, needs_layout_passes=False),
        scratch_types=[
            pltpu.VMEM_SHARED((NROW, D), F32),
            pltpu.VMEM((C,), jnp.int32), pltpu.VMEM((C,), jnp.int32),
            pltpu.VMEM((C,), jnp.int32), pltpu.VMEM((C,), jnp.int32),
            pltpu.VMEM((C, D), F32), pltpu.VMEM((C, D), F32),
            pltpu.VMEM((C, D), F32), pltpu.VMEM((C, D), F32),
            pltpu.VMEM((C, ED), F32), pltpu.VMEM((C, ED), F32),
            pltpu.VMEM((C, ED), F32), pltpu.VMEM((C, ED), F32),
            pltpu.VMEM((C, ED), F32), pltpu.VMEM((C, ED), F32),
            pltpu.SemaphoreType.DMA, pltpu.SemaphoreType.DMA,
        ],
    )
    def k(hp_hbm, ep_hbm, h1_hbm, h2_hbm, ee_hbm, src_hbm, dst_hbm, z_hbm,
          agg_hbm, enew_hbm, agg_sh,
          isrc0, isrc1, idst0, idst1, gat0, gat1, epb0, epb1,
          g10, g11, g20, g21, eeb0, eeb1, sem0, sem1):
        cid = lax.axis_index("c")
        sid = lax.axis_index("s")
        wid = cid * NS + sid
        isrc = [isrc0, isrc1]
        idst = [idst0, idst1]
        gat = [gat0, gat1]
        epb = [epb0, epb1]
        g1b = [g10, g11]
        g2b = [g20, g21]
        eeb = [eeb0, eeb1]
        sem = [sem0, sem1]

        pltpu.sync_copy(z_hbm.at[pl.ds(sid * NPS, NPS)],
                        agg_sh.at[pl.ds(sid * NPS, NPS)])
        plsc.subcore_barrier()

        def issue(c, b):
            base = wid * EPW + c * C
            pltpu.sync_copy(src_hbm.at[pl.ds(base, C)], isrc[b])
            pltpu.sync_copy(dst_hbm.at[pl.ds(base, C)], idst[b])
            pltpu.async_copy(hp_hbm.at[isrc[b]], gat[b], sem[b])
            pltpu.async_copy(ep_hbm.at[pl.ds(base, C)], epb[b], sem[b])
            pltpu.async_copy(h1_hbm.at[isrc[b]], g1b[b], sem[b])
            pltpu.async_copy(h2_hbm.at[idst[b]], g2b[b], sem[b])
            pltpu.async_copy(ee_hbm.at[pl.ds(base, C)], eeb[b], sem[b])

        def drain_in(b):
            pltpu.make_async_copy(hp_hbm.at[isrc[b]], gat[b], sem[b]).wait()
            pltpu.make_async_copy(ep_hbm.at[pl.ds(0, C)], epb[b], sem[b]).wait()
            pltpu.make_async_copy(h1_hbm.at[isrc[b]], g1b[b], sem[b]).wait()
            pltpu.make_async_copy(h2_hbm.at[idst[b]], g2b[b], sem[b]).wait()
            pltpu.make_async_copy(ee_hbm.at[pl.ds(0, C)], eeb[b], sem[b]).wait()

        def flush(c, b):
            def row(r, _):
                for j in range(D // 16):
                    sl = pl.ds(j * 16, 16)
                    epb[b][r, sl] = jnp.maximum(gat[b][r, sl] + epb[b][r, sl], 0.0)
                eeb[b][r, :] = jnp.maximum(
                    eeb[b][r, :] + g1b[b][r, :] + g2b[b][r, :], 0.0)
                return 0
            lax.fori_loop(0, C, row, 0)
            base = wid * EPW + c * C
            pltpu.sync_copy(epb[b], agg_sh.at[idst[b]], add=True)
            pltpu.sync_copy(eeb[b], enew_hbm.at[pl.ds(base, C)])

        issue(0, 0)

        def pair(i, _):
            for t in range(2):
                c = 2 * i + t

                @pl.when(c + 1 < NCHUNK)
                def _():
                    issue(c + 1, 1 - t)
                drain_in(t)
                flush(c, t)
            return 0
        lax.fori_loop(0, NCHUNK // 2, pair, 0)

        plsc.subcore_barrier()
        pltpu.sync_copy(agg_sh.at[pl.ds(sid * NPS, NPS)],
                        agg_hbm.at[pl.ds(cid * NROW + sid * NPS, NPS)])

    return k(hp, ep, h1, h2, ee, srcv, dstv, zeros_nd)


def _sc_edge_head(a, b, srcv, dstv):
    @functools.partial(
        pl.kernel,
        out_type=_sds((E, 64)),
        mesh=_MESH,
        compiler_params=pltpu.CompilerParams(use_tc_tiling_on_sc=False, needs_layout_passes=False),
        scratch_types=[
            pltpu.VMEM((C,), jnp.int32), pltpu.VMEM((C,), jnp.int32),
            pltpu.VMEM((C,), jnp.int32), pltpu.VMEM((C,), jnp.int32),
            pltpu.VMEM((C, 64), F32), pltpu.VMEM((C, 64), F32),
            pltpu.VMEM((C, 64), F32), pltpu.VMEM((C, 64), F32),
            pltpu.SemaphoreType.DMA, pltpu.SemaphoreType.DMA,
        ],
    )
    def k(a_hbm, b_hbm, src_hbm, dst_hbm, out_hbm,
          isrc0, isrc1, idst0, idst1, ga0, ga1, gb0, gb1, sem0, sem1):
        cid = lax.axis_index("c")
        sid = lax.axis_index("s")
        wid = cid * NS + sid
        isrc = [isrc0, isrc1]
        idst = [idst0, idst1]
        ga = [ga0, ga1]
        gb = [gb0, gb1]
        sem = [sem0, sem1]

        def issue(c, bb):
            base = wid * EPW + c * C
            pltpu.sync_copy(src_hbm.at[pl.ds(base, C)], isrc[bb])
            pltpu.sync_copy(dst_hbm.at[pl.ds(base, C)], idst[bb])
            pltpu.async_copy(a_hbm.at[isrc[bb]], ga[bb], sem[bb])
            pltpu.async_copy(b_hbm.at[idst[bb]], gb[bb], sem[bb])

        def drain_in(bb):
            pltpu.make_async_copy(a_hbm.at[isrc[bb]], ga[bb], sem[bb]).wait()
            pltpu.make_async_copy(b_hbm.at[idst[bb]], gb[bb], sem[bb]).wait()

        def flush(c, bb):
            def row(r, _):
                for j in range(4):
                    sl = pl.ds(j * 16, 16)
                    ga[bb][r, sl] = jnp.maximum(ga[bb][r, sl] + gb[bb][r, sl], 0.0)
                return 0
            lax.fori_loop(0, C, row, 0)
            base = wid * EPW + c * C
            pltpu.sync_copy(ga[bb], out_hbm.at[pl.ds(base, C)])

        issue(0, 0)

        def pair(i, _):
            for t in range(2):
                c = 2 * i + t

                @pl.when(c + 1 < NCHUNK)
                def _():
                    issue(c + 1, 1 - t)
                drain_in(t)
                flush(c, t)
            return 0
        lax.fori_loop(0, NCHUNK // 2, pair, 0)

    return k(a, b, srcv, dstv)


def kernel(h, e, edge_index, params):
    src = edge_index[0]
    dst = edge_index[1]

    deg_parts = _sc_degree(dst)
    deg_all = deg_parts.reshape(NW, NPAD, 1)[:, :N, :]

    zeros_nd = jnp.zeros((NROW, D), F32)

    for lp in params["layers"]:
        pw = lp["P"]["W"]
        qw = lp["Q"]["W"]
        ww = lp["W"]["W"]
        pw1, pw2 = pw[:D], pw[D:]
        qw1, qw2 = qw[:D], qw[D:]
        we, wh1, wh2 = ww[:ED], ww[ED:ED + D], ww[ED + D:]
        pb = lp["P"]["b"].reshape(1, D)
        qb = lp["Q"]["b"].reshape(1, D)
        wb = lp["W"]["b"].reshape(1, ED)

        hp, h1, h2, hq = _tc_node_pre(h, pw1, wh1, wh2, qw1)
        ep, ee = _tc_edge_pre(e, pw2, pb, we, wb)
        agg2, e_new = _sc_layer(hp, ep, h1, h2, ee, src, dst, zeros_nd)
        h = _tc_node_update(hq, agg2[:N], agg2[NROW:NROW + N], deg_all,
                            qw2, qb)
        e = e_new

    eh = params["edge_head"]
    nh = params["node_head"]
    a0 = eh["l0"]["W"][:D]
    b0c = eh["l0"]["W"][D:]
    bb0 = eh["l0"]["b"].reshape(1, 64)
    a, b, node_pred = _tc_heads_node(
        h, a0, b0c, bb0,
        nh["l0"]["W"], nh["l0"]["b"].reshape(1, 64),
        nh["l1"]["W"], nh["l1"]["b"].reshape(1, 1))

    r = _sc_edge_head(a, b, src, dst)
    edge_pred = _tc_edge_head(r, eh["l1"]["W"], eh["l1"]["b"].reshape(1, 1))

    return (h, edge_pred.reshape(E), node_pred.reshape(N))


# speedup vs baseline: 1.4906x; 1.1099x over previous
import functools

import jax
import jax.numpy as jnp
from jax import lax
from jax.experimental import pallas as pl
from jax.experimental.pallas import tpu as pltpu
from jax.experimental.pallas import tpu_sc as plsc

N = 10000
E = 320000
D = 128
ED = 16
NC = 2
NS = 16
NW = NC * NS
EPW = E // NW
C = 80
NCHUNK = EPW // C
CD = 80
NCHUNK_D = EPW // CD
NROW = 10112
NPS = NROW // NS
NPAD = 10240

F32 = jnp.float32


def _sds(shape, dtype=F32):
    return jax.ShapeDtypeStruct(shape, dtype)


def _tc_node_pre(h, pw1, wh1, wh2, qw1):
    def body(h_ref, pw1_ref, wh1_ref, wh2_ref, qw1_ref, hp, h1, h2, hq):
        hb = h_ref[...]
        hp[...] = jnp.dot(hb, pw1_ref[...], preferred_element_type=F32)
        h1[...] = jnp.dot(hb, wh1_ref[...], preferred_element_type=F32)
        h2[...] = jnp.dot(hb, wh2_ref[...], preferred_element_type=F32)
        hq[...] = jnp.dot(hb, qw1_ref[...], preferred_element_type=F32)

    g = 10
    blk = N // g
    full = lambda s: pl.BlockSpec(s, lambda i: (0, 0))
    return pl.pallas_call(
        body,
        grid=(g,),
        in_specs=[
            pl.BlockSpec((blk, D), lambda i: (i, 0)),
            full((D, D)), full((D, ED)), full((D, ED)), full((D, D)),
        ],
        out_specs=[
            pl.BlockSpec((blk, D), lambda i: (i, 0)),
            pl.BlockSpec((blk, ED), lambda i: (i, 0)),
            pl.BlockSpec((blk, ED), lambda i: (i, 0)),
            pl.BlockSpec((blk, D), lambda i: (i, 0)),
        ],
        out_shape=[_sds((N, D)), _sds((N, ED)), _sds((N, ED)), _sds((N, D))],
    )(h, pw1, wh1, wh2, qw1)


def _tc_edge_pre(e, pw2, pb, we, wb):
    def body(e_ref, pw2_ref, pb_ref, we_ref, wb_ref, ep, ee):
        eb = e_ref[...]
        ep[...] = jnp.dot(eb, pw2_ref[...], preferred_element_type=F32) + pb_ref[...]
        ee[...] = jnp.dot(eb, we_ref[...], preferred_element_type=F32) + wb_ref[...]

    g = 80
    blk = E // g
    full = lambda s: pl.BlockSpec(s, lambda i: (0, 0))
    return pl.pallas_call(
        body,
        grid=(g,),
        in_specs=[
            pl.BlockSpec((blk, ED), lambda i: (i, 0)),
            full((ED, D)), full((1, D)), full((ED, ED)), full((1, ED)),
        ],
        out_specs=[
            pl.BlockSpec((blk, D), lambda i: (i, 0)),
            pl.BlockSpec((blk, ED), lambda i: (i, 0)),
        ],
        out_shape=[_sds((E, D)), _sds((E, ED))],
    )(e, pw2, pb, we, wb)


def _tc_node_update(hq, agga, aggb, deg_all, qw2, qb):
    def body(hq_ref, aa_ref, ab_ref, deg_ref, qw2_ref, qb_ref, out):
        deg = jnp.sum(deg_ref[...], axis=0)
        agg = (aa_ref[...] + ab_ref[...]) / deg
        out[...] = jnp.maximum(
            hq_ref[...]
            + jnp.dot(agg, qw2_ref[...], preferred_element_type=F32)
            + qb_ref[...], 0.0)

    g = 10
    blk = N // g
    full = lambda s: pl.BlockSpec(s, lambda i: (0, 0))
    return pl.pallas_call(
        body,
        grid=(g,),
        in_specs=[
            pl.BlockSpec((blk, D), lambda i: (i, 0)),
            pl.BlockSpec((blk, D), lambda i: (i, 0)),
            pl.BlockSpec((blk, D), lambda i: (i, 0)),
            pl.BlockSpec((NW, blk, 1), lambda i: (0, i, 0)),
            full((D, D)), full((1, D)),
        ],
        out_specs=pl.BlockSpec((blk, D), lambda i: (i, 0)),
        out_shape=_sds((N, D)),
    )(hq, agga, aggb, deg_all, qw2, qb)


def _tc_heads_node(h, a0, b0c, bb0, n0w, n0b, n1w, n1b):
    def body(h_ref, a0_ref, b0_ref, bb0_ref, n0w_ref, n0b_ref, n1w_ref,
             n1b_ref, a_out, b_out, np_out):
        hb = h_ref[...]
        a_out[...] = jnp.dot(hb, a0_ref[...], preferred_element_type=F32) + bb0_ref[...]
        b_out[...] = jnp.dot(hb, b0_ref[...], preferred_element_type=F32)
        hid = jnp.maximum(
            jnp.dot(hb, n0w_ref[...], preferred_element_type=F32) + n0b_ref[...], 0.0)
        np_out[...] = jnp.dot(hid, n1w_ref[...], preferred_element_type=F32) + n1b_ref[...]

    g = 10
    blk = N // g
    full = lambda s: pl.BlockSpec(s, lambda i: (0, 0))
    return pl.pallas_call(
        body,
        grid=(g,),
        in_specs=[
            pl.BlockSpec((blk, D), lambda i: (i, 0)),
            full((D, 64)), full((D, 64)), full((1, 64)),
            full((D, 64)), full((1, 64)), full((64, 1)), full((1, 1)),
        ],
        out_specs=[
            pl.BlockSpec((blk, 64), lambda i: (i, 0)),
            pl.BlockSpec((blk, 64), lambda i: (i, 0)),
            pl.BlockSpec((blk, 1), lambda i: (i, 0)),
        ],
        out_shape=[_sds((N, 64)), _sds((N, 64)), _sds((N, 1))],
    )(h, a0, b0c, bb0, n0w, n0b, n1w, n1b)


def _tc_edge_head(r, w1, b1):
    def body(r_ref, w1_ref, b1_ref, out):
        out[...] = jnp.dot(r_ref[...], w1_ref[...], preferred_element_type=F32) + b1_ref[...]

    g = 80
    blk = E // g
    full = lambda s: pl.BlockSpec(s, lambda i: (0, 0))
    return pl.pallas_call(
        body,
        grid=(g,),
        in_specs=[
            pl.BlockSpec((blk, 64), lambda i: (i, 0)),
            full((64, 1)), full((1, 1)),
        ],
        out_specs=pl.BlockSpec((blk, 1), lambda i: (i, 0)),
        out_shape=_sds((E, 1)),
    )(r, w1, b1)


_MESH = plsc.VectorSubcoreMesh(core_axis_name="c", subcore_axis_name="s")


def _sc_degree(dst):
    @functools.partial(
        pl.kernel,
        out_type=_sds((NW * (NPAD // 16), 16)),
        mesh=_MESH,
        compiler_params=pltpu.CompilerParams(use_tc_tiling_on_sc=False, needs_layout_passes=False),
        scratch_types=[
            pltpu.VMEM((NPAD // 16, 16), F32),
            pltpu.VMEM((CD,), jnp.int32),
        ],
    )
    def k(dst_hbm, out_hbm, hist_v, idx_v):
        cid = lax.axis_index("c")
        sid = lax.axis_index("s")
        wid = cid * NS + sid
        zero16 = jnp.zeros((16,), F32)

        def zbody(r, _):
            hist_v[r, :] = zero16
            return 0
        lax.fori_loop(0, NPAD // 16, zbody, 0)

        ones16 = jnp.ones((16,), F32)

        def chunk(c, _):
            base = wid * EPW + c * CD
            pltpu.sync_copy(dst_hbm.at[pl.ds(base, CD)], idx_v)
            for kk in range(CD // 16):
                iv = idx_v[pl.ds(kk * 16, 16)]
                rows = lax.shift_right_logical(iv, 4)
                cols = lax.bitwise_and(iv, 15)
                plsc.addupdate_scatter(hist_v, [rows, cols], ones16)
            return 0
        lax.fori_loop(0, NCHUNK_D, chunk, 0)

        pltpu.sync_copy(hist_v, out_hbm.at[pl.ds(wid * (NPAD // 16), NPAD // 16)])

    return k(dst)


def _sc_layer(hp, ep, h1, h2, ee, srcv, dstv, zeros_nd):
    @functools.partial(
        pl.kernel,
        out_type=(_sds((2 * NROW, D)), _sds((E, ED))),
        mesh=_MESH,
        compiler_params=pltpu.CompilerParams(use_tc_tiling_on_sc=False, needs_layout_passes=False),
        scratch_types=[
            pltpu.VMEM_SHARED((NROW, D), F32),
            pltpu.VMEM((C,), jnp.int32), pltpu.VMEM((C,), jnp.int32),
            pltpu.VMEM((C,), jnp.int32), pltpu.VMEM((C,), jnp.int32),
            pltpu.VMEM((C, D), F32), pltpu.VMEM((C, D), F32),
            pltpu.VMEM((C, D), F32), pltpu.VMEM((C, D), F32),
            pltpu.VMEM((C, ED), F32), pltpu.VMEM((C, ED), F32),
            pltpu.VMEM((C, ED), F32), pltpu.VMEM((C, ED), F32),
            pltpu.VMEM((C, ED), F32), pltpu.VMEM((C, ED), F32),
            pltpu.SemaphoreType.DMA, pltpu.SemaphoreType.DMA,
        ],
    )
    def k(hp_hbm, ep_hbm, h1_hbm, h2_hbm, ee_hbm, src_hbm, dst_hbm, z_hbm,
          agg_hbm, enew_hbm, agg_sh,
          isrc0, isrc1, idst0, idst1, gat0, gat1, epb0, epb1,
          g10, g11, g20, g21, eeb0, eeb1, sem0, sem1):
        cid = lax.axis_index("c")
        sid = lax.axis_index("s")
        wid = cid * NS + sid
        isrc = [isrc0, isrc1]
        idst = [idst0, idst1]
        gat = [gat0, gat1]
        epb = [epb0, epb1]
        g1b = [g10, g11]
        g2b = [g20, g21]
        eeb = [eeb0, eeb1]
        sem = [sem0, sem1]

        pltpu.sync_copy(z_hbm.at[pl.ds(sid * NPS, NPS)],
                        agg_sh.at[pl.ds(sid * NPS, NPS)])
        plsc.subcore_barrier()

        def issue(c, b):
            base = wid * EPW + c * C
            pltpu.sync_copy(src_hbm.at[pl.ds(base, C)], isrc[b])
            pltpu.sync_copy(dst_hbm.at[pl.ds(base, C)], idst[b])
            pltpu.async_copy(hp_hbm.at[isrc[b]], gat[b], sem[b])
            pltpu.async_copy(ep_hbm.at[pl.ds(base, C)], epb[b], sem[b])
            pltpu.async_copy(h1_hbm.at[isrc[b]], g1b[b], sem[b])
            pltpu.async_copy(h2_hbm.at[idst[b]], g2b[b], sem[b])
            pltpu.async_copy(ee_hbm.at[pl.ds(base, C)], eeb[b], sem[b])

        def drain_in(b):
            pltpu.make_async_copy(hp_hbm.at[isrc[b]], gat[b], sem[b]).wait()
            pltpu.make_async_copy(ep_hbm.at[pl.ds(0, C)], epb[b], sem[b]).wait()
            pltpu.make_async_copy(h1_hbm.at[isrc[b]], g1b[b], sem[b]).wait()
            pltpu.make_async_copy(h2_hbm.at[idst[b]], g2b[b], sem[b]).wait()
            pltpu.make_async_copy(ee_hbm.at[pl.ds(0, C)], eeb[b], sem[b]).wait()

        def flush(c, b):
            def row(r, _):
                for j in range(D // 16):
                    sl = pl.ds(j * 16, 16)
                    epb[b][r, sl] = jnp.maximum(gat[b][r, sl] + epb[b][r, sl], 0.0)
                eeb[b][r, :] = jnp.maximum(
                    eeb[b][r, :] + g1b[b][r, :] + g2b[b][r, :], 0.0)
                return 0
            lax.fori_loop(0, C, row, 0)
            base = wid * EPW + c * C
            pltpu.sync_copy(epb[b], agg_sh.at[idst[b]], add=True)
            pltpu.sync_copy(eeb[b], enew_hbm.at[pl.ds(base, C)])

        issue(0, 0)

        def pair(i, _):
            for t in range(2):
                c = 2 * i + t

                @pl.when(c + 1 < NCHUNK)
                def _():
                    issue(c + 1, 1 - t)
                drain_in(t)
                flush(c, t)
            return 0
        lax.fori_loop(0, NCHUNK // 2, pair, 0)

        if NCHUNK % 2 == 1:
            drain_in(0)
            flush(NCHUNK - 1, 0)

        plsc.subcore_barrier()
        pltpu.sync_copy(agg_sh.at[pl.ds(sid * NPS, NPS)],
                        agg_hbm.at[pl.ds(cid * NROW + sid * NPS, NPS)])

    return k(hp, ep, h1, h2, ee, srcv, dstv, zeros_nd)


def _sc_edge_head(a, b, srcv, dstv):
    @functools.partial(
        pl.kernel,
        out_type=_sds((E, 64)),
        mesh=_MESH,
        compiler_params=pltpu.CompilerParams(use_tc_tiling_on_sc=False, needs_layout_passes=False),
        scratch_types=[
            pltpu.VMEM((C,), jnp.int32), pltpu.VMEM((C,), jnp.int32),
            pltpu.VMEM((C,), jnp.int32), pltpu.VMEM((C,), jnp.int32),
            pltpu.VMEM((C, 64), F32), pltpu.VMEM((C, 64), F32),
            pltpu.VMEM((C, 64), F32), pltpu.VMEM((C, 64), F32),
            pltpu.SemaphoreType.DMA, pltpu.SemaphoreType.DMA,
        ],
    )
    def k(a_hbm, b_hbm, src_hbm, dst_hbm, out_hbm,
          isrc0, isrc1, idst0, idst1, ga0, ga1, gb0, gb1, sem0, sem1):
        cid = lax.axis_index("c")
        sid = lax.axis_index("s")
        wid = cid * NS + sid
        isrc = [isrc0, isrc1]
        idst = [idst0, idst1]
        ga = [ga0, ga1]
        gb = [gb0, gb1]
        sem = [sem0, sem1]

        def issue(c, bb):
            base = wid * EPW + c * C
            pltpu.sync_copy(src_hbm.at[pl.ds(base, C)], isrc[bb])
            pltpu.sync_copy(dst_hbm.at[pl.ds(base, C)], idst[bb])
            pltpu.async_copy(a_hbm.at[isrc[bb]], ga[bb], sem[bb])
            pltpu.async_copy(b_hbm.at[idst[bb]], gb[bb], sem[bb])

        def drain_in(bb):
            pltpu.make_async_copy(a_hbm.at[isrc[bb]], ga[bb], sem[bb]).wait()
            pltpu.make_async_copy(b_hbm.at[idst[bb]], gb[bb], sem[bb]).wait()

        def flush(c, bb):
            def row(r, _):
                for j in range(4):
                    sl = pl.ds(j * 16, 16)
                    ga[bb][r, sl] = jnp.maximum(ga[bb][r, sl] + gb[bb][r, sl], 0.0)
                return 0
            lax.fori_loop(0, C, row, 0)
            base = wid * EPW + c * C
            pltpu.sync_copy(ga[bb], out_hbm.at[pl.ds(base, C)])

        issue(0, 0)

        def pair(i, _):
            for t in range(2):
                c = 2 * i + t

                @pl.when(c + 1 < NCHUNK)
                def _():
                    issue(c + 1, 1 - t)
                drain_in(t)
                flush(c, t)
            return 0
        lax.fori_loop(0, NCHUNK // 2, pair, 0)

        if NCHUNK % 2 == 1:
            drain_in(0)
            flush(NCHUNK - 1, 0)

    return k(a, b, srcv, dstv)


def kernel(h, e, edge_index, params):
    src = edge_index[0]
    dst = edge_index[1]

    deg_parts = _sc_degree(dst)
    deg_all = deg_parts.reshape(NW, NPAD, 1)[:, :N, :]

    zeros_nd = jnp.zeros((NROW, D), F32)

    for lp in params["layers"]:
        pw = lp["P"]["W"]
        qw = lp["Q"]["W"]
        ww = lp["W"]["W"]
        pw1, pw2 = pw[:D], pw[D:]
        qw1, qw2 = qw[:D], qw[D:]
        we, wh1, wh2 = ww[:ED], ww[ED:ED + D], ww[ED + D:]
        pb = lp["P"]["b"].reshape(1, D)
        qb = lp["Q"]["b"].reshape(1, D)
        wb = lp["W"]["b"].reshape(1, ED)

        hp, h1, h2, hq = _tc_node_pre(h, pw1, wh1, wh2, qw1)
        ep, ee = _tc_edge_pre(e, pw2, pb, we, wb)
        agg2, e_new = _sc_layer(hp, ep, h1, h2, ee, src, dst, zeros_nd)
        h = _tc_node_update(hq, agg2[:N], agg2[NROW:NROW + N], deg_all,
                            qw2, qb)
        e = e_new

    eh = params["edge_head"]
    nh = params["node_head"]
    a0 = eh["l0"]["W"][:D]
    b0c = eh["l0"]["W"][D:]
    bb0 = eh["l0"]["b"].reshape(1, 64)
    a, b, node_pred = _tc_heads_node(
        h, a0, b0c, bb0,
        nh["l0"]["W"], nh["l0"]["b"].reshape(1, 64),
        nh["l1"]["W"], nh["l1"]["b"].reshape(1, 1))

    r = _sc_edge_head(a, b, src, dst)
    edge_pred = _tc_edge_head(r, eh["l1"]["W"], eh["l1"]["b"].reshape(1, 1))

    return (h, edge_pred.reshape(E), node_pred.reshape(N))


# speedup vs baseline: 2.0276x; 1.3603x over previous
import functools

import jax
import jax.numpy as jnp
from jax import lax
from jax.experimental import pallas as pl
from jax.experimental.pallas import tpu as pltpu
from jax.experimental.pallas import tpu_sc as plsc

N = 10000
E = 320000
D = 128
ED = 16
NC = 2
NS = 16
NW = NC * NS
EPW = E // NW
C = 80
NCHUNK = EPW // C
CD = 80
NCHUNK_D = EPW // CD
NROW = 10112
NPS = NROW // NS
NPAD = 10240

F32 = jnp.float32


def _sds(shape, dtype=F32):
    return jax.ShapeDtypeStruct(shape, dtype)


def _tc_node_pre(h, pw1, wh1, wh2, qw1):
    def body(h_ref, pw1_ref, wh1_ref, wh2_ref, qw1_ref, hp, h1, h2, hq):
        hb = h_ref[...]
        hp[...] = jnp.dot(hb, pw1_ref[...], preferred_element_type=F32)
        h1[...] = jnp.dot(hb, wh1_ref[...], preferred_element_type=F32)
        h2[...] = jnp.dot(hb, wh2_ref[...], preferred_element_type=F32)
        hq[...] = jnp.dot(hb, qw1_ref[...], preferred_element_type=F32)

    g = 10
    blk = N // g
    full = lambda s: pl.BlockSpec(s, lambda i: (0, 0))
    return pl.pallas_call(
        body,
        grid=(g,),
        in_specs=[
            pl.BlockSpec((blk, D), lambda i: (i, 0)),
            full((D, D)), full((D, ED)), full((D, ED)), full((D, D)),
        ],
        out_specs=[
            pl.BlockSpec((blk, D), lambda i: (i, 0)),
            pl.BlockSpec((blk, ED), lambda i: (i, 0)),
            pl.BlockSpec((blk, ED), lambda i: (i, 0)),
            pl.BlockSpec((blk, D), lambda i: (i, 0)),
        ],
        out_shape=[_sds((N, D)), _sds((N, ED)), _sds((N, ED)), _sds((N, D))],
    )(h, pw1, wh1, wh2, qw1)


def _tc_edge_pre(e, pw2, pb, we, wb):
    def body(e_ref, pw2_ref, pb_ref, we_ref, wb_ref, ep, ee):
        eb = e_ref[...]
        ep[...] = jnp.dot(eb, pw2_ref[...], preferred_element_type=F32) + pb_ref[...]
        ee[...] = jnp.dot(eb, we_ref[...], preferred_element_type=F32) + wb_ref[...]

    g = 80
    blk = E // g
    full = lambda s: pl.BlockSpec(s, lambda i: (0, 0))
    return pl.pallas_call(
        body,
        grid=(g,),
        in_specs=[
            pl.BlockSpec((blk, ED), lambda i: (i, 0)),
            full((ED, D)), full((1, D)), full((ED, ED)), full((1, ED)),
        ],
        out_specs=[
            pl.BlockSpec((blk, D), lambda i: (i, 0)),
            pl.BlockSpec((blk, ED), lambda i: (i, 0)),
        ],
        out_shape=[_sds((E, D)), _sds((E, ED))],
    )(e, pw2, pb, we, wb)


def _tc_node_update(hq, agga, aggb, deg_all, qw2, qb):
    def body(hq_ref, aa_ref, ab_ref, deg_ref, qw2_ref, qb_ref, out):
        deg = jnp.sum(deg_ref[...], axis=0)
        agg = (aa_ref[...] + ab_ref[...]) / deg
        out[...] = jnp.maximum(
            hq_ref[...]
            + jnp.dot(agg, qw2_ref[...], preferred_element_type=F32)
            + qb_ref[...], 0.0)

    g = 10
    blk = N // g
    full = lambda s: pl.BlockSpec(s, lambda i: (0, 0))
    return pl.pallas_call(
        body,
        grid=(g,),
        in_specs=[
            pl.BlockSpec((blk, D), lambda i: (i, 0)),
            pl.BlockSpec((blk, D), lambda i: (i, 0)),
            pl.BlockSpec((blk, D), lambda i: (i, 0)),
            pl.BlockSpec((NW, blk, 1), lambda i: (0, i, 0)),
            full((D, D)), full((1, D)),
        ],
        out_specs=pl.BlockSpec((blk, D), lambda i: (i, 0)),
        out_shape=_sds((N, D)),
    )(hq, agga, aggb, deg_all, qw2, qb)


def _tc_heads_node(h, a0, b0c, bb0, n0w, n0b, n1w, n1b):
    def body(h_ref, a0_ref, b0_ref, bb0_ref, n0w_ref, n0b_ref, n1w_ref,
             n1b_ref, a_out, b_out, np_out):
        hb = h_ref[...]
        a_out[...] = jnp.dot(hb, a0_ref[...], preferred_element_type=F32) + bb0_ref[...]
        b_out[...] = jnp.dot(hb, b0_ref[...], preferred_element_type=F32)
        hid = jnp.maximum(
            jnp.dot(hb, n0w_ref[...], preferred_element_type=F32) + n0b_ref[...], 0.0)
        np_out[...] = jnp.dot(hid, n1w_ref[...], preferred_element_type=F32) + n1b_ref[...]

    g = 10
    blk = N // g
    full = lambda s: pl.BlockSpec(s, lambda i: (0, 0))
    return pl.pallas_call(
        body,
        grid=(g,),
        in_specs=[
            pl.BlockSpec((blk, D), lambda i: (i, 0)),
            full((D, 64)), full((D, 64)), full((1, 64)),
            full((D, 64)), full((1, 64)), full((64, 1)), full((1, 1)),
        ],
        out_specs=[
            pl.BlockSpec((blk, 64), lambda i: (i, 0)),
            pl.BlockSpec((blk, 64), lambda i: (i, 0)),
            pl.BlockSpec((blk, 1), lambda i: (i, 0)),
        ],
        out_shape=[_sds((N, 64)), _sds((N, 64)), _sds((N, 1))],
    )(h, a0, b0c, bb0, n0w, n0b, n1w, n1b)


def _tc_edge_head(r, w1, b1):
    def body(r_ref, w1_ref, b1_ref, out):
        out[...] = jnp.dot(r_ref[...], w1_ref[...], preferred_element_type=F32) + b1_ref[...]

    g = 80
    blk = E // g
    full = lambda s: pl.BlockSpec(s, lambda i: (0, 0))
    return pl.pallas_call(
        body,
        grid=(g,),
        in_specs=[
            pl.BlockSpec((blk, 64), lambda i: (i, 0)),
            full((64, 1)), full((1, 1)),
        ],
        out_specs=pl.BlockSpec((blk, 1), lambda i: (i, 0)),
        out_shape=_sds((E, 1)),
    )(r, w1, b1)


_MESH = plsc.VectorSubcoreMesh(core_axis_name="c", subcore_axis_name="s")


def _sc_degree(dst):
    @functools.partial(
        pl.kernel,
        out_type=_sds((NW * (NPAD // 16), 16)),
        mesh=_MESH,
        compiler_params=pltpu.CompilerParams(use_tc_tiling_on_sc=False, needs_layout_passes=False),
        scratch_types=[
            pltpu.VMEM((NPAD // 16, 16), F32),
            pltpu.VMEM((CD,), jnp.int32),
        ],
    )
    def k(dst_hbm, out_hbm, hist_v, idx_v):
        cid = lax.axis_index("c")
        sid = lax.axis_index("s")
        wid = cid * NS + sid
        zero16 = jnp.zeros((16,), F32)

        def zbody(r, _):
            hist_v[r, :] = zero16
            return 0
        lax.fori_loop(0, NPAD // 16, zbody, 0)

        ones16 = jnp.ones((16,), F32)

        def chunk(c, _):
            base = wid * EPW + c * CD
            pltpu.sync_copy(dst_hbm.at[pl.ds(base, CD)], idx_v)
            for kk in range(CD // 16):
                iv = idx_v[pl.ds(kk * 16, 16)]
                rows = lax.shift_right_logical(iv, 4)
                cols = lax.bitwise_and(iv, 15)
                plsc.addupdate_scatter(hist_v, [rows, cols], ones16)
            return 0
        lax.fori_loop(0, NCHUNK_D, chunk, 0)

        pltpu.sync_copy(hist_v, out_hbm.at[pl.ds(wid * (NPAD // 16), NPAD // 16)])

    return k(dst)


def _sc_layer(hp, ep, h1, h2, ee, srcv, dstv, zeros_nd):
    @functools.partial(
        pl.kernel,
        out_type=(_sds((2 * NROW, D)), _sds((E, ED))),
        mesh=_MESH,
        compiler_params=pltpu.CompilerParams(use_tc_tiling_on_sc=False, needs_layout_passes=False),
        scratch_types=[
            pltpu.VMEM_SHARED((NROW, D), F32),
            pltpu.VMEM((C,), jnp.int32), pltpu.VMEM((C,), jnp.int32),
            pltpu.VMEM((C,), jnp.int32), pltpu.VMEM((C,), jnp.int32),
            pltpu.VMEM((C, D), F32), pltpu.VMEM((C, D), F32),
            pltpu.VMEM((C, D), F32), pltpu.VMEM((C, D), F32),
            pltpu.VMEM((C, ED), F32), pltpu.VMEM((C, ED), F32),
            pltpu.VMEM((C, ED), F32), pltpu.VMEM((C, ED), F32),
            pltpu.VMEM((C, ED), F32), pltpu.VMEM((C, ED), F32),
            pltpu.SemaphoreType.DMA, pltpu.SemaphoreType.DMA,
        ],
    )
    def k(hp_hbm, ep_hbm, h1_hbm, h2_hbm, ee_hbm, src_hbm, dst_hbm, z_hbm,
          agg_hbm, enew_hbm, agg_sh,
          isrc0, isrc1, idst0, idst1, gat0, gat1, epb0, epb1,
          g10, g11, g20, g21, eeb0, eeb1, sem0, sem1):
        cid = lax.axis_index("c")
        sid = lax.axis_index("s")
        wid = cid * NS + sid
        isrc = [isrc0, isrc1]
        idst = [idst0, idst1]
        gat = [gat0, gat1]
        epb = [epb0, epb1]
        g1b = [g10, g11]
        g2b = [g20, g21]
        eeb = [eeb0, eeb1]
        sem = [sem0, sem1]

        pltpu.sync_copy(z_hbm.at[pl.ds(sid * NPS, NPS)],
                        agg_sh.at[pl.ds(sid * NPS, NPS)])
        plsc.subcore_barrier()

        def issue(c, b):
            base = wid * EPW + c * C
            pltpu.sync_copy(src_hbm.at[pl.ds(base, C)], isrc[b])
            pltpu.sync_copy(dst_hbm.at[pl.ds(base, C)], idst[b])
            pltpu.async_copy(hp_hbm.at[isrc[b]], gat[b], sem[b])
            pltpu.async_copy(ep_hbm.at[pl.ds(base, C)], epb[b], sem[b])
            pltpu.async_copy(h1_hbm.at[isrc[b]], g1b[b], sem[b])
            pltpu.async_copy(h2_hbm.at[idst[b]], g2b[b], sem[b])
            pltpu.async_copy(ee_hbm.at[pl.ds(base, C)], eeb[b], sem[b])

        def drain_in(b):
            pltpu.make_async_copy(hp_hbm.at[isrc[b]], gat[b], sem[b]).wait()
            pltpu.make_async_copy(ep_hbm.at[pl.ds(0, C)], epb[b], sem[b]).wait()
            pltpu.make_async_copy(h1_hbm.at[isrc[b]], g1b[b], sem[b]).wait()
            pltpu.make_async_copy(h2_hbm.at[idst[b]], g2b[b], sem[b]).wait()
            pltpu.make_async_copy(ee_hbm.at[pl.ds(0, C)], eeb[b], sem[b]).wait()

        def flush(c, b):
            @plsc.parallel_loop(0, C, unroll=2)
            def row(r):
                for j in range(D // 16):
                    sl = pl.ds(j * 16, 16)
                    epb[b][r, sl] = jnp.maximum(gat[b][r, sl] + epb[b][r, sl], 0.0)
                eeb[b][r, :] = jnp.maximum(
                    eeb[b][r, :] + g1b[b][r, :] + g2b[b][r, :], 0.0)
            base = wid * EPW + c * C
            pltpu.sync_copy(epb[b], agg_sh.at[idst[b]], add=True)
            pltpu.sync_copy(eeb[b], enew_hbm.at[pl.ds(base, C)])

        issue(0, 0)

        def pair(i, _):
            for t in range(2):
                c = 2 * i + t

                @pl.when(c + 1 < NCHUNK)
                def _():
                    issue(c + 1, 1 - t)
                drain_in(t)
                flush(c, t)
            return 0
        lax.fori_loop(0, NCHUNK // 2, pair, 0)

        if NCHUNK % 2 == 1:
            drain_in(0)
            flush(NCHUNK - 1, 0)

        plsc.subcore_barrier()
        pltpu.sync_copy(agg_sh.at[pl.ds(sid * NPS, NPS)],
                        agg_hbm.at[pl.ds(cid * NROW + sid * NPS, NPS)])

    return k(hp, ep, h1, h2, ee, srcv, dstv, zeros_nd)


def _sc_edge_head(a, b, srcv, dstv):
    @functools.partial(
        pl.kernel,
        out_type=_sds((E, 64)),
        mesh=_MESH,
        compiler_params=pltpu.CompilerParams(use_tc_tiling_on_sc=False, needs_layout_passes=False),
        scratch_types=[
            pltpu.VMEM((C,), jnp.int32), pltpu.VMEM((C,), jnp.int32),
            pltpu.VMEM((C,), jnp.int32), pltpu.VMEM((C,), jnp.int32),
            pltpu.VMEM((C, 64), F32), pltpu.VMEM((C, 64), F32),
            pltpu.VMEM((C, 64), F32), pltpu.VMEM((C, 64), F32),
            pltpu.SemaphoreType.DMA, pltpu.SemaphoreType.DMA,
        ],
    )
    def k(a_hbm, b_hbm, src_hbm, dst_hbm, out_hbm,
          isrc0, isrc1, idst0, idst1, ga0, ga1, gb0, gb1, sem0, sem1):
        cid = lax.axis_index("c")
        sid = lax.axis_index("s")
        wid = cid * NS + sid
        isrc = [isrc0, isrc1]
        idst = [idst0, idst1]
        ga = [ga0, ga1]
        gb = [gb0, gb1]
        sem = [sem0, sem1]

        def issue(c, bb):
            base = wid * EPW + c * C
            pltpu.sync_copy(src_hbm.at[pl.ds(base, C)], isrc[bb])
            pltpu.sync_copy(dst_hbm.at[pl.ds(base, C)], idst[bb])
            pltpu.async_copy(a_hbm.at[isrc[bb]], ga[bb], sem[bb])
            pltpu.async_copy(b_hbm.at[idst[bb]], gb[bb], sem[bb])

        def drain_in(bb):
            pltpu.make_async_copy(a_hbm.at[isrc[bb]], ga[bb], sem[bb]).wait()
            pltpu.make_async_copy(b_hbm.at[idst[bb]], gb[bb], sem[bb]).wait()

        def flush(c, bb):
            @plsc.parallel_loop(0, C, unroll=2)
            def row(r):
                for j in range(4):
                    sl = pl.ds(j * 16, 16)
                    ga[bb][r, sl] = jnp.maximum(ga[bb][r, sl] + gb[bb][r, sl], 0.0)
            base = wid * EPW + c * C
            pltpu.sync_copy(ga[bb], out_hbm.at[pl.ds(base, C)])

        issue(0, 0)

        def pair(i, _):
            for t in range(2):
                c = 2 * i + t

                @pl.when(c + 1 < NCHUNK)
                def _():
                    issue(c + 1, 1 - t)
                drain_in(t)
                flush(c, t)
            return 0
        lax.fori_loop(0, NCHUNK // 2, pair, 0)

        if NCHUNK % 2 == 1:
            drain_in(0)
            flush(NCHUNK - 1, 0)

    return k(a, b, srcv, dstv)


def kernel(h, e, edge_index, params):
    src = edge_index[0]
    dst = edge_index[1]

    deg_parts = _sc_degree(dst)
    deg_all = deg_parts.reshape(NW, NPAD, 1)[:, :N, :]

    zeros_nd = jnp.zeros((NROW, D), F32)

    for lp in params["layers"]:
        pw = lp["P"]["W"]
        qw = lp["Q"]["W"]
        ww = lp["W"]["W"]
        pw1, pw2 = pw[:D], pw[D:]
        qw1, qw2 = qw[:D], qw[D:]
        we, wh1, wh2 = ww[:ED], ww[ED:ED + D], ww[ED + D:]
        pb = lp["P"]["b"].reshape(1, D)
        qb = lp["Q"]["b"].reshape(1, D)
        wb = lp["W"]["b"].reshape(1, ED)

        hp, h1, h2, hq = _tc_node_pre(h, pw1, wh1, wh2, qw1)
        ep, ee = _tc_edge_pre(e, pw2, pb, we, wb)
        agg2, e_new = _sc_layer(hp, ep, h1, h2, ee, src, dst, zeros_nd)
        h = _tc_node_update(hq, agg2[:N], agg2[NROW:NROW + N], deg_all,
                            qw2, qb)
        e = e_new

    eh = params["edge_head"]
    nh = params["node_head"]
    a0 = eh["l0"]["W"][:D]
    b0c = eh["l0"]["W"][D:]
    bb0 = eh["l0"]["b"].reshape(1, 64)
    a, b, node_pred = _tc_heads_node(
        h, a0, b0c, bb0,
        nh["l0"]["W"], nh["l0"]["b"].reshape(1, 64),
        nh["l1"]["W"], nh["l1"]["b"].reshape(1, 1))

    r = _sc_edge_head(a, b, src, dst)
    edge_pred = _tc_edge_head(r, eh["l1"]["W"], eh["l1"]["b"].reshape(1, 1))

    return (h, edge_pred.reshape(E), node_pred.reshape(N))


# speedup vs baseline: 2.0337x; 1.0030x over previous
import functools

import jax
import jax.numpy as jnp
from jax import lax
from jax.experimental import pallas as pl
from jax.experimental.pallas import tpu as pltpu
from jax.experimental.pallas import tpu_sc as plsc

N = 10000
E = 320000
D = 128
ED = 16
NC = 2
NS = 16
NW = NC * NS
EPW = E // NW
C = 80
NCHUNK = EPW // C
CD = 80
NCHUNK_D = EPW // CD
NROW = 10112
NPS = NROW // NS
NPAD = 10240

F32 = jnp.float32


def _sds(shape, dtype=F32):
    return jax.ShapeDtypeStruct(shape, dtype)


def _tc_node_pre(h, pw1, wh1, wh2, qw1):
    def body(h_ref, pw1_ref, wh1_ref, wh2_ref, qw1_ref, hp, h1, h2, hq):
        hb = h_ref[...]
        hp[...] = jnp.dot(hb, pw1_ref[...], preferred_element_type=F32)
        h1[...] = jnp.dot(hb, wh1_ref[...], preferred_element_type=F32)
        h2[...] = jnp.dot(hb, wh2_ref[...], preferred_element_type=F32)
        hq[...] = jnp.dot(hb, qw1_ref[...], preferred_element_type=F32)

    g = 10
    blk = N // g
    full = lambda s: pl.BlockSpec(s, lambda i: (0, 0))
    return pl.pallas_call(
        body,
        grid=(g,),
        in_specs=[
            pl.BlockSpec((blk, D), lambda i: (i, 0)),
            full((D, D)), full((D, ED)), full((D, ED)), full((D, D)),
        ],
        out_specs=[
            pl.BlockSpec((blk, D), lambda i: (i, 0)),
            pl.BlockSpec((blk, ED), lambda i: (i, 0)),
            pl.BlockSpec((blk, ED), lambda i: (i, 0)),
            pl.BlockSpec((blk, D), lambda i: (i, 0)),
        ],
        out_shape=[_sds((N, D)), _sds((N, ED)), _sds((N, ED)), _sds((N, D))],
    )(h, pw1, wh1, wh2, qw1)


def _tc_edge_pre(e, pw2, pb, we, wb):
    def body(e_ref, pw2_ref, pb_ref, we_ref, wb_ref, ep, ee):
        eb = e_ref[...]
        ep[...] = jnp.dot(eb, pw2_ref[...], preferred_element_type=F32) + pb_ref[...]
        ee[...] = jnp.dot(eb, we_ref[...], preferred_element_type=F32) + wb_ref[...]

    g = 80
    blk = E // g
    full = lambda s: pl.BlockSpec(s, lambda i: (0, 0))
    return pl.pallas_call(
        body,
        grid=(g,),
        in_specs=[
            pl.BlockSpec((blk, ED), lambda i: (i, 0)),
            full((ED, D)), full((1, D)), full((ED, ED)), full((1, ED)),
        ],
        out_specs=[
            pl.BlockSpec((blk, D), lambda i: (i, 0)),
            pl.BlockSpec((blk, ED), lambda i: (i, 0)),
        ],
        out_shape=[_sds((E, D)), _sds((E, ED))],
    )(e, pw2, pb, we, wb)


def _tc_node_update(hq, agga, aggb, deg_all, qw2, qb):
    def body(hq_ref, aa_ref, ab_ref, deg_ref, qw2_ref, qb_ref, out):
        deg = jnp.sum(deg_ref[...], axis=0)
        agg = (aa_ref[...] + ab_ref[...]) / deg
        out[...] = jnp.maximum(
            hq_ref[...]
            + jnp.dot(agg, qw2_ref[...], preferred_element_type=F32)
            + qb_ref[...], 0.0)

    g = 10
    blk = N // g
    full = lambda s: pl.BlockSpec(s, lambda i: (0, 0))
    return pl.pallas_call(
        body,
        grid=(g,),
        in_specs=[
            pl.BlockSpec((blk, D), lambda i: (i, 0)),
            pl.BlockSpec((blk, D), lambda i: (i, 0)),
            pl.BlockSpec((blk, D), lambda i: (i, 0)),
            pl.BlockSpec((NW, blk, 1), lambda i: (0, i, 0)),
            full((D, D)), full((1, D)),
        ],
        out_specs=pl.BlockSpec((blk, D), lambda i: (i, 0)),
        out_shape=_sds((N, D)),
    )(hq, agga, aggb, deg_all, qw2, qb)


def _tc_heads_node(h, a0, b0c, bb0, n0w, n0b, n1w, n1b):
    def body(h_ref, a0_ref, b0_ref, bb0_ref, n0w_ref, n0b_ref, n1w_ref,
             n1b_ref, a_out, b_out, np_out):
        hb = h_ref[...]
        a_out[...] = jnp.dot(hb, a0_ref[...], preferred_element_type=F32) + bb0_ref[...]
        b_out[...] = jnp.dot(hb, b0_ref[...], preferred_element_type=F32)
        hid = jnp.maximum(
            jnp.dot(hb, n0w_ref[...], preferred_element_type=F32) + n0b_ref[...], 0.0)
        np_out[...] = jnp.dot(hid, n1w_ref[...], preferred_element_type=F32) + n1b_ref[...]

    g = 10
    blk = N // g
    full = lambda s: pl.BlockSpec(s, lambda i: (0, 0))
    return pl.pallas_call(
        body,
        grid=(g,),
        in_specs=[
            pl.BlockSpec((blk, D), lambda i: (i, 0)),
            full((D, 64)), full((D, 64)), full((1, 64)),
            full((D, 64)), full((1, 64)), full((64, 1)), full((1, 1)),
        ],
        out_specs=[
            pl.BlockSpec((blk, 64), lambda i: (i, 0)),
            pl.BlockSpec((blk, 64), lambda i: (i, 0)),
            pl.BlockSpec((blk, 1), lambda i: (i, 0)),
        ],
        out_shape=[_sds((N, 64)), _sds((N, 64)), _sds((N, 1))],
    )(h, a0, b0c, bb0, n0w, n0b, n1w, n1b)


def _tc_edge_head(r, w1, b1):
    def body(r_ref, w1_ref, b1_ref, out):
        out[...] = jnp.dot(r_ref[...], w1_ref[...], preferred_element_type=F32) + b1_ref[...]

    g = 80
    blk = E // g
    full = lambda s: pl.BlockSpec(s, lambda i: (0, 0))
    return pl.pallas_call(
        body,
        grid=(g,),
        in_specs=[
            pl.BlockSpec((blk, 64), lambda i: (i, 0)),
            full((64, 1)), full((1, 1)),
        ],
        out_specs=pl.BlockSpec((blk, 1), lambda i: (i, 0)),
        out_shape=_sds((E, 1)),
    )(r, w1, b1)


_MESH = plsc.VectorSubcoreMesh(core_axis_name="c", subcore_axis_name="s")


def _sc_degree(dst):
    @functools.partial(
        pl.kernel,
        out_type=_sds((NW * (NPAD // 16), 16)),
        mesh=_MESH,
        compiler_params=pltpu.CompilerParams(use_tc_tiling_on_sc=False, needs_layout_passes=False),
        scratch_types=[
            pltpu.VMEM((NPAD // 16, 16), F32),
            pltpu.VMEM((CD,), jnp.int32),
        ],
    )
    def k(dst_hbm, out_hbm, hist_v, idx_v):
        cid = lax.axis_index("c")
        sid = lax.axis_index("s")
        wid = cid * NS + sid
        zero16 = jnp.zeros((16,), F32)

        def zbody(r, _):
            hist_v[r, :] = zero16
            return 0
        lax.fori_loop(0, NPAD // 16, zbody, 0)

        ones16 = jnp.ones((16,), F32)

        def chunk(c, _):
            base = wid * EPW + c * CD
            pltpu.sync_copy(dst_hbm.at[pl.ds(base, CD)], idx_v)
            for kk in range(CD // 16):
                iv = idx_v[pl.ds(kk * 16, 16)]
                rows = lax.shift_right_logical(iv, 4)
                cols = lax.bitwise_and(iv, 15)
                plsc.addupdate_scatter(hist_v, [rows, cols], ones16)
            return 0
        lax.fori_loop(0, NCHUNK_D, chunk, 0)

        pltpu.sync_copy(hist_v, out_hbm.at[pl.ds(wid * (NPAD // 16), NPAD // 16)])

    return k(dst)


def _sc_layer(hp, ep, h1, h2, ee, srcv, dstv, zeros_nd):
    @functools.partial(
        pl.kernel,
        out_type=(_sds((2 * NROW, D)), _sds((E, ED))),
        mesh=_MESH,
        compiler_params=pltpu.CompilerParams(use_tc_tiling_on_sc=False, needs_layout_passes=False),
        scratch_types=[
            pltpu.VMEM_SHARED((NROW, D), F32),
            pltpu.VMEM((C,), jnp.int32), pltpu.VMEM((C,), jnp.int32),
            pltpu.VMEM((C,), jnp.int32), pltpu.VMEM((C,), jnp.int32),
            pltpu.VMEM((C, D), F32), pltpu.VMEM((C, D), F32),
            pltpu.VMEM((C, D), F32), pltpu.VMEM((C, D), F32),
            pltpu.VMEM((C, ED), F32), pltpu.VMEM((C, ED), F32),
            pltpu.VMEM((C, ED), F32), pltpu.VMEM((C, ED), F32),
            pltpu.VMEM((C, ED), F32), pltpu.VMEM((C, ED), F32),
            pltpu.SemaphoreType.DMA, pltpu.SemaphoreType.DMA,
        ],
    )
    def k(hp_hbm, ep_hbm, h1_hbm, h2_hbm, ee_hbm, src_hbm, dst_hbm, z_hbm,
          agg_hbm, enew_hbm, agg_sh,
          isrc0, isrc1, idst0, idst1, gat0, gat1, epb0, epb1,
          g10, g11, g20, g21, eeb0, eeb1, sem0, sem1):
        cid = lax.axis_index("c")
        sid = lax.axis_index("s")
        wid = cid * NS + sid
        isrc = [isrc0, isrc1]
        idst = [idst0, idst1]
        gat = [gat0, gat1]
        epb = [epb0, epb1]
        g1b = [g10, g11]
        g2b = [g20, g21]
        eeb = [eeb0, eeb1]
        sem = [sem0, sem1]

        pltpu.sync_copy(z_hbm.at[pl.ds(sid * NPS, NPS)],
                        agg_sh.at[pl.ds(sid * NPS, NPS)])
        plsc.subcore_barrier()

        def issue(c, b):
            base = wid * EPW + c * C
            pltpu.sync_copy(src_hbm.at[pl.ds(base, C)], isrc[b])
            pltpu.sync_copy(dst_hbm.at[pl.ds(base, C)], idst[b])
            pltpu.async_copy(hp_hbm.at[isrc[b]], gat[b], sem[b])
            pltpu.async_copy(ep_hbm.at[pl.ds(base, C)], epb[b], sem[b])
            pltpu.async_copy(h1_hbm.at[isrc[b]], g1b[b], sem[b])
            pltpu.async_copy(h2_hbm.at[idst[b]], g2b[b], sem[b])
            pltpu.async_copy(ee_hbm.at[pl.ds(base, C)], eeb[b], sem[b])

        def drain_in(b):
            pltpu.make_async_copy(hp_hbm.at[isrc[b]], gat[b], sem[b]).wait()
            pltpu.make_async_copy(ep_hbm.at[pl.ds(0, C)], epb[b], sem[b]).wait()
            pltpu.make_async_copy(h1_hbm.at[isrc[b]], g1b[b], sem[b]).wait()
            pltpu.make_async_copy(h2_hbm.at[idst[b]], g2b[b], sem[b]).wait()
            pltpu.make_async_copy(ee_hbm.at[pl.ds(0, C)], eeb[b], sem[b]).wait()

        def flush(c, b):
            @plsc.parallel_loop(0, C, unroll=4)
            def row(r):
                for j in range(D // 16):
                    sl = pl.ds(j * 16, 16)
                    epb[b][r, sl] = jnp.maximum(gat[b][r, sl] + epb[b][r, sl], 0.0)
                eeb[b][r, :] = jnp.maximum(
                    eeb[b][r, :] + g1b[b][r, :] + g2b[b][r, :], 0.0)
            base = wid * EPW + c * C
            pltpu.sync_copy(epb[b], agg_sh.at[idst[b]], add=True)
            pltpu.sync_copy(eeb[b], enew_hbm.at[pl.ds(base, C)])

        issue(0, 0)

        def pair(i, _):
            for t in range(2):
                c = 2 * i + t

                @pl.when(c + 1 < NCHUNK)
                def _():
                    issue(c + 1, 1 - t)
                drain_in(t)
                flush(c, t)
            return 0
        lax.fori_loop(0, NCHUNK // 2, pair, 0)

        if NCHUNK % 2 == 1:
            drain_in(0)
            flush(NCHUNK - 1, 0)

        plsc.subcore_barrier()
        pltpu.sync_copy(agg_sh.at[pl.ds(sid * NPS, NPS)],
                        agg_hbm.at[pl.ds(cid * NROW + sid * NPS, NPS)])

    return k(hp, ep, h1, h2, ee, srcv, dstv, zeros_nd)


def _sc_edge_head(a, b, srcv, dstv):
    @functools.partial(
        pl.kernel,
        out_type=_sds((E, 64)),
        mesh=_MESH,
        compiler_params=pltpu.CompilerParams(use_tc_tiling_on_sc=False, needs_layout_passes=False),
        scratch_types=[
            pltpu.VMEM((C,), jnp.int32), pltpu.VMEM((C,), jnp.int32),
            pltpu.VMEM((C,), jnp.int32), pltpu.VMEM((C,), jnp.int32),
            pltpu.VMEM((C, 64), F32), pltpu.VMEM((C, 64), F32),
            pltpu.VMEM((C, 64), F32), pltpu.VMEM((C, 64), F32),
            pltpu.SemaphoreType.DMA, pltpu.SemaphoreType.DMA,
        ],
    )
    def k(a_hbm, b_hbm, src_hbm, dst_hbm, out_hbm,
          isrc0, isrc1, idst0, idst1, ga0, ga1, gb0, gb1, sem0, sem1):
        cid = lax.axis_index("c")
        sid = lax.axis_index("s")
        wid = cid * NS + sid
        isrc = [isrc0, isrc1]
        idst = [idst0, idst1]
        ga = [ga0, ga1]
        gb = [gb0, gb1]
        sem = [sem0, sem1]

        def issue(c, bb):
            base = wid * EPW + c * C
            pltpu.sync_copy(src_hbm.at[pl.ds(base, C)], isrc[bb])
            pltpu.sync_copy(dst_hbm.at[pl.ds(base, C)], idst[bb])
            pltpu.async_copy(a_hbm.at[isrc[bb]], ga[bb], sem[bb])
            pltpu.async_copy(b_hbm.at[idst[bb]], gb[bb], sem[bb])

        def drain_in(bb):
            pltpu.make_async_copy(a_hbm.at[isrc[bb]], ga[bb], sem[bb]).wait()
            pltpu.make_async_copy(b_hbm.at[idst[bb]], gb[bb], sem[bb]).wait()

        def flush(c, bb):
            @plsc.parallel_loop(0, C, unroll=4)
            def row(r):
                for j in range(4):
                    sl = pl.ds(j * 16, 16)
                    ga[bb][r, sl] = jnp.maximum(ga[bb][r, sl] + gb[bb][r, sl], 0.0)
            base = wid * EPW + c * C
            pltpu.sync_copy(ga[bb], out_hbm.at[pl.ds(base, C)])

        issue(0, 0)

        def pair(i, _):
            for t in range(2):
                c = 2 * i + t

                @pl.when(c + 1 < NCHUNK)
                def _():
                    issue(c + 1, 1 - t)
                drain_in(t)
                flush(c, t)
            return 0
        lax.fori_loop(0, NCHUNK // 2, pair, 0)

        if NCHUNK % 2 == 1:
            drain_in(0)
            flush(NCHUNK - 1, 0)

    return k(a, b, srcv, dstv)


def kernel(h, e, edge_index, params):
    src = edge_index[0]
    dst = edge_index[1]

    deg_parts = _sc_degree(dst)
    deg_all = deg_parts.reshape(NW, NPAD, 1)[:, :N, :]

    zeros_nd = jnp.zeros((NROW, D), F32)

    for lp in params["layers"]:
        pw = lp["P"]["W"]
        qw = lp["Q"]["W"]
        ww = lp["W"]["W"]
        pw1, pw2 = pw[:D], pw[D:]
        qw1, qw2 = qw[:D], qw[D:]
        we, wh1, wh2 = ww[:ED], ww[ED:ED + D], ww[ED + D:]
        pb = lp["P"]["b"].reshape(1, D)
        qb = lp["Q"]["b"].reshape(1, D)
        wb = lp["W"]["b"].reshape(1, ED)

        hp, h1, h2, hq = _tc_node_pre(h, pw1, wh1, wh2, qw1)
        ep, ee = _tc_edge_pre(e, pw2, pb, we, wb)
        agg2, e_new = _sc_layer(hp, ep, h1, h2, ee, src, dst, zeros_nd)
        h = _tc_node_update(hq, agg2[:N], agg2[NROW:NROW + N], deg_all,
                            qw2, qb)
        e = e_new

    eh = params["edge_head"]
    nh = params["node_head"]
    a0 = eh["l0"]["W"][:D]
    b0c = eh["l0"]["W"][D:]
    bb0 = eh["l0"]["b"].reshape(1, 64)
    a, b, node_pred = _tc_heads_node(
        h, a0, b0c, bb0,
        nh["l0"]["W"], nh["l0"]["b"].reshape(1, 64),
        nh["l1"]["W"], nh["l1"]["b"].reshape(1, 1))

    r = _sc_edge_head(a, b, src, dst)
    edge_pred = _tc_edge_head(r, eh["l1"]["W"], eh["l1"]["b"].reshape(1, 1))

    return (h, edge_pred.reshape(E), node_pred.reshape(N))
